# baseline (speedup 1.0000x reference)
import functools

import jax
import jax.numpy as jnp
from jax.experimental import pallas as pl
from jax.experimental.pallas import tpu as pltpu

N = 10000
E = 320000
D = 128
H = 256

ROW_BLK = 1000


def _pre_body(x_ref, wskip_ref, wfs_ref, wlin_ref, wf_ref, bf_ref,
              skip_ref, xl_ref, f2_ref):
    xb = x_ref[...]
    fs = jnp.dot(xb, wfs_ref[...], preferred_element_type=jnp.float32)
    beta_s = fs[:, :D]
    gamma_s = fs[:, D:]
    sk = jnp.dot(xb, wskip_ref[...], preferred_element_type=jnp.float32)
    skip_ref[...] = jnp.maximum(gamma_s * sk + beta_s, 0.0)
    xl_ref[...] = jnp.dot(xb, wlin_ref[...], preferred_element_type=jnp.float32)
    f2_ref[...] = jnp.dot(xb, wf_ref[...], preferred_element_type=jnp.float32) + bf_ref[...]


def _pre(x, W_skip, W_fs, W_lin, W_f, b_f):
    grid = (N // ROW_BLK,)
    row_spec = pl.BlockSpec((ROW_BLK, D), lambda i: (i, 0))
    full = lambda shape: pl.BlockSpec(shape, lambda i: (0, 0))
    return pl.pallas_call(
        _pre_body,
        grid=grid,
        in_specs=[
            row_spec,
            full((D, D)), full((D, 2 * D)), full((D, D)), full((D, 2 * D)),
            full((1, 2 * D)),
        ],
        out_specs=[
            row_spec,
            row_spec,
            pl.BlockSpec((ROW_BLK, 2 * D), lambda i: (i, 0)),
        ],
        out_shape=[
            jax.ShapeDtypeStruct((N, D), jnp.float32),
            jax.ShapeDtypeStruct((N, D), jnp.float32),
            jax.ShapeDtypeStruct((N, 2 * D), jnp.float32),
        ],
    )(x, W_skip, W_fs, W_lin, W_f, b_f.reshape(1, 2 * D))


def _mlp_body(skip_ref, agg_ref, deg_ref, w1_ref, b1_ref, w2_ref, b2_ref,
              w3_ref, b3_ref, wr_ref, br_ref, h_ref, sums_ref):
    deg = jnp.maximum(deg_ref[...], 1.0)
    h = skip_ref[...] + agg_ref[...] / deg
    h = jnp.dot(h, w1_ref[...], preferred_element_type=jnp.float32) + b1_ref[...]
    h = jnp.where(h > 0, h, 0.01 * h)
    h = jnp.dot(h, w2_ref[...], preferred_element_type=jnp.float32) + b2_ref[...]
    h = jnp.where(h > 0, h, 0.01 * h)
    h = jnp.dot(h, w3_ref[...], preferred_element_type=jnp.float32) + b3_ref[...]
    h = jnp.where(h > 0, h, 0.01 * h)
    h = jnp.dot(h, wr_ref[...], preferred_element_type=jnp.float32) + br_ref[...]
    h_ref[...] = h

    @pl.when(pl.program_id(0) == 0)
    def _():
        sums_ref[...] = jnp.zeros_like(sums_ref)

    s1 = jnp.sum(h, axis=0, keepdims=True)
    s2 = jnp.sum(h * h, axis=0, keepdims=True)
    sums_ref[...] += jnp.concatenate(
        [s1, s2, jnp.zeros((6, D), jnp.float32)], axis=0)


def _mlp(skip, agg, deg, W1, b1, W2, b2, W3, b3, Wr, br):
    grid = (N // ROW_BLK,)
    row_spec = pl.BlockSpec((ROW_BLK, D), lambda i: (i, 0))
    full = lambda shape: pl.BlockSpec(shape, lambda i: (0, 0))
    return pl.pallas_call(
        _mlp_body,
        grid=grid,
        in_specs=[
            row_spec, row_spec,
            pl.BlockSpec((ROW_BLK, 1), lambda i: (i, 0)),
            full((D, H)), full((1, H)), full((H, H)), full((1, H)),
            full((H, H)), full((1, H)), full((H, D)), full((1, D)),
        ],
        out_specs=[row_spec, full((8, D))],
        out_shape=[
            jax.ShapeDtypeStruct((N, D), jnp.float32),
            jax.ShapeDtypeStruct((8, D), jnp.float32),
        ],
    )(skip, agg, deg, W1, b1.reshape(1, H), W2, b2.reshape(1, H),
      W3, b3.reshape(1, H), Wr, br.reshape(1, D))


def _norm_body(h_ref, sums_ref, gnw_ref, gnb_ref, gnms_ref, out_ref):
    s1 = sums_ref[0:1, :]
    s2 = sums_ref[1:2, :]
    inv_n = 1.0 / N
    mean = s1 * inv_n
    c = gnms_ref[...] * mean
    var = s2 * inv_n - 2.0 * c * mean + c * c
    scale = gnw_ref[...] * jax.lax.rsqrt(var + 1e-5)
    out_ref[...] = (h_ref[...] - c) * scale + gnb_ref[...]


def _norm(h, sums, gn_w, gn_b, gn_ms):
    grid = (N // ROW_BLK,)
    row_spec = pl.BlockSpec((ROW_BLK, D), lambda i: (i, 0))
    full = lambda shape: pl.BlockSpec(shape, lambda i: (0, 0))
    return pl.pallas_call(
        _norm_body,
        grid=grid,
        in_specs=[row_spec, full((8, D)), full((1, D)), full((1, D)),
                  full((1, D))],
        out_specs=row_spec,
        out_shape=jax.ShapeDtypeStruct((N, D), jnp.float32),
    )(h, sums, gn_w.reshape(1, D), gn_b.reshape(1, D), gn_ms.reshape(1, D))


def kernel(x, edge_index, W_skip, W_fs, W_lin, W_f, b_f, W1, b1, W2, b2,
           W3, b3, Wr, br, gn_w, gn_b, gn_ms):
    skip, xl, f2 = _pre(x, W_skip, W_fs, W_lin, W_f, b_f)
    src = edge_index[0]
    dst = edge_index[1]
    beta = f2[:, :D]
    gamma = f2[:, D:]
    msg = jnp.maximum(gamma[dst] * xl[src] + beta[dst], 0.0)
    agg = jax.ops.segment_sum(msg, dst, num_segments=N)
    deg = jax.ops.segment_sum(jnp.ones((E,), jnp.float32), dst, num_segments=N)
    h, sums = _mlp(skip, agg, deg.reshape(N, 1), W1, b1, W2, b2, W3, b3, Wr, br)
    return _norm(h, sums, gn_w, gn_b, gn_ms)


# speedup vs baseline: 2.5691x; 2.5691x over previous
import functools

import jax
import jax.numpy as jnp
from jax import lax
from jax.experimental import pallas as pl
from jax.experimental.pallas import tpu as pltpu
from jax.experimental.pallas import tpu_sc as plsc

N = 10000
E = 320000
D = 128
H = 256

ROW_BLK = 1000

NC = 2
NS = 16
NW = NC * NS
RNG = 320
NPAD = NW * RNG
TRASH = RNG
ACC_ROWS = RNG + 1
LIST_CAP = 16384
SCAN = 4000
K = 128
PACK_SHIFT = 9


def _pre_body(x_ref, wskip_ref, wfs_ref, wlin_ref, wf_ref, bf_ref,
              skip_ref, xl_ref, f2_ref):
    xb = x_ref[...]
    fs = jnp.dot(xb, wfs_ref[...], preferred_element_type=jnp.float32)
    beta_s = fs[:, :D]
    gamma_s = fs[:, D:]
    sk = jnp.dot(xb, wskip_ref[...], preferred_element_type=jnp.float32)
    skip_ref[...] = jnp.maximum(gamma_s * sk + beta_s, 0.0)
    xl_ref[...] = jnp.dot(xb, wlin_ref[...], preferred_element_type=jnp.float32)
    f2_ref[...] = jnp.dot(xb, wf_ref[...], preferred_element_type=jnp.float32) + bf_ref[...]


def _pre(x, W_skip, W_fs, W_lin, W_f, b_f):
    grid = (N // ROW_BLK,)
    row_spec = pl.BlockSpec((ROW_BLK, D), lambda i: (i, 0))
    full = lambda shape: pl.BlockSpec(shape, lambda i: (0, 0))
    return pl.pallas_call(
        _pre_body,
        grid=grid,
        in_specs=[
            row_spec,
            full((D, D)), full((D, 2 * D)), full((D, D)), full((D, 2 * D)),
            full((1, 2 * D)),
        ],
        out_specs=[
            row_spec,
            row_spec,
            pl.BlockSpec((ROW_BLK, 2 * D), lambda i: (i, 0)),
        ],
        out_shape=[
            jax.ShapeDtypeStruct((N, D), jnp.float32),
            jax.ShapeDtypeStruct((N, D), jnp.float32),
            jax.ShapeDtypeStruct((N, 2 * D), jnp.float32),
        ],
    )(x, W_skip, W_fs, W_lin, W_f, b_f.reshape(1, 2 * D))


def _edge_body(xl_hbm, f2_hbm, esrc_hbm, edst_hbm, agg_out, deg_out,
               srcbuf, dstbuf, list_ref,
               src_idx, dst_idx, xl_buf, f2_buf,
               acc, deg_flat, sem1, sem2):
    wid = lax.axis_index("s") * NC + lax.axis_index("c")
    lo = wid * RNG
    hi = jnp.minimum(lo + RNG, N)

    def _zrow(i, _):
        for j in range(D // 16):
            acc[i, pl.ds(j * 16, 16)] = jnp.zeros((16,), jnp.float32)
        deg_flat[pl.ds(i * 16, 16)] = jnp.zeros((16,), jnp.float32)
        return 0
    lax.fori_loop(0, ACC_ROWS, _zrow, 0)

    trash_fill = jnp.full((16,), TRASH, jnp.int32)
    def _fill(i, _):
        list_ref[pl.ds(i * 16, 16)] = trash_fill
        return 0
    lax.fori_loop(0, LIST_CAP // 16 + 1, _fill, 0)

    lanes = lax.iota(jnp.int32, 16)
    ones16 = jnp.full((16,), 1.0, jnp.float32)

    def _scan_chunk(c, cnt):
        off = c * SCAN
        cp1 = pltpu.async_copy(esrc_hbm.at[pl.ds(off, SCAN)], srcbuf, sem1)
        cp2 = pltpu.async_copy(edst_hbm.at[pl.ds(off, SCAN)], dstbuf, sem2)
        cp1.wait()
        cp2.wait()

        def _scan_vreg(v, cnt):
            d = dstbuf[pl.ds(v * 16, 16)]
            s = srcbuf[pl.ds(v * 16, 16)]
            m = (d >= lo) & (d < hi)
            ld = jnp.where(m, d - lo, TRASH)
            plsc.addupdate_scatter(deg_flat, [ld * 16 + lanes], ones16, mask=m)
            rank = plsc.cumsum(m.astype(jnp.int32))
            idx = cnt + rank - 1
            packed = (s << PACK_SHIFT) | ld
            plsc.store_scatter(list_ref, [idx], packed, mask=m)
            return cnt + plsc.all_reduce_population_count(m)

        return lax.fori_loop(0, SCAN // 16, _scan_vreg, cnt)

    cnt = lax.fori_loop(0, E // SCAN, _scan_chunk,
                        jnp.zeros((16,), jnp.int32))
    n_edges = cnt[0]
    n_chunks = (n_edges + (K - 1)) // K

    def _proc_chunk(c, _):
        base = c * K
        for v in range(K // 16):
            p = list_ref[pl.ds(base + v * 16, 16)]
            ld = p & ((1 << PACK_SHIFT) - 1)
            src_idx[pl.ds(v * 16, 16)] = p >> PACK_SHIFT
            dst_idx[pl.ds(v * 16, 16)] = jnp.minimum(ld + lo, N - 1)
        g1 = pltpu.async_copy(xl_hbm.at[src_idx], xl_buf, sem1)
        g2 = pltpu.async_copy(f2_hbm.at[dst_idx], f2_buf, sem2)
        g1.wait()
        g2.wait()

        def _edge(i, _):
            pv = list_ref[pl.ds(base + i, 16)]
            ld = pv[0] & ((1 << PACK_SHIFT) - 1)
            for j in range(D // 16):
                xv = xl_buf[i, pl.ds(j * 16, 16)]
                bv = f2_buf[i, pl.ds(j * 16, 16)]
                gv = f2_buf[i, pl.ds(D + j * 16, 16)]
                acc[ld, pl.ds(j * 16, 16)] = acc[ld, pl.ds(j * 16, 16)] + \
                    jnp.maximum(gv * xv + bv, 0.0)
            return 0

        lax.fori_loop(0, K, _edge, 0)
        return 0

    lax.fori_loop(0, n_chunks, _proc_chunk, 0)

    pltpu.sync_copy(acc.at[pl.ds(0, RNG)], agg_out.at[pl.ds(lo, RNG)])
    pltpu.sync_copy(deg_flat.at[pl.ds(0, RNG * 16)],
                    deg_out.at[pl.ds(lo * 16, RNG * 16)])


def _edge(xl, f2, esrc, edst):
    mesh = plsc.VectorSubcoreMesh(core_axis_name="c", subcore_axis_name="s",
                                  num_cores=NC, num_subcores=NS)
    return pl.kernel(
        _edge_body,
        out_type=[
            jax.ShapeDtypeStruct((NPAD, D), jnp.float32),
            jax.ShapeDtypeStruct((NPAD * 16,), jnp.float32),
        ],
        mesh=mesh,
        compiler_params=pltpu.CompilerParams(needs_layout_passes=False),
        scratch_types=[
            pltpu.VMEM((SCAN,), jnp.int32),
            pltpu.VMEM((SCAN,), jnp.int32),
            pltpu.VMEM((LIST_CAP + 16,), jnp.int32),
            pltpu.VMEM((K,), jnp.int32),
            pltpu.VMEM((K,), jnp.int32),
            pltpu.VMEM((K, D), jnp.float32),
            pltpu.VMEM((K, 2 * D), jnp.float32),
            pltpu.VMEM((ACC_ROWS, D), jnp.float32),
            pltpu.VMEM((ACC_ROWS * 16,), jnp.float32),
            pltpu.SemaphoreType.DMA,
            pltpu.SemaphoreType.DMA,
        ],
    )(xl, f2, esrc, edst)


def _mlp_body(skip_ref, agg_ref, deg_ref, w1_ref, b1_ref, w2_ref, b2_ref,
              w3_ref, b3_ref, wr_ref, br_ref, h_ref, sums_ref):
    deg = jnp.sum(deg_ref[...], axis=1, keepdims=True)
    deg = jnp.maximum(deg, 1.0)
    h = skip_ref[...] + agg_ref[...] / deg
    h = jnp.dot(h, w1_ref[...], preferred_element_type=jnp.float32) + b1_ref[...]
    h = jnp.where(h > 0, h, 0.01 * h)
    h = jnp.dot(h, w2_ref[...], preferred_element_type=jnp.float32) + b2_ref[...]
    h = jnp.where(h > 0, h, 0.01 * h)
    h = jnp.dot(h, w3_ref[...], preferred_element_type=jnp.float32) + b3_ref[...]
    h = jnp.where(h > 0, h, 0.01 * h)
    h = jnp.dot(h, wr_ref[...], preferred_element_type=jnp.float32) + br_ref[...]
    h_ref[...] = h

    @pl.when(pl.program_id(0) == 0)
    def _():
        sums_ref[...] = jnp.zeros_like(sums_ref)

    s1 = jnp.sum(h, axis=0, keepdims=True)
    s2 = jnp.sum(h * h, axis=0, keepdims=True)
    sums_ref[...] += jnp.concatenate(
        [s1, s2, jnp.zeros((6, D), jnp.float32)], axis=0)


def _mlp(skip, agg, deg2d, W1, b1, W2, b2, W3, b3, Wr, br):
    grid = (N // ROW_BLK,)
    row_spec = pl.BlockSpec((ROW_BLK, D), lambda i: (i, 0))
    full = lambda shape: pl.BlockSpec(shape, lambda i: (0, 0))
    return pl.pallas_call(
        _mlp_body,
        grid=grid,
        in_specs=[
            row_spec, row_spec,
            pl.BlockSpec((ROW_BLK, 16), lambda i: (i, 0)),
            full((D, H)), full((1, H)), full((H, H)), full((1, H)),
            full((H, H)), full((1, H)), full((H, D)), full((1, D)),
        ],
        out_specs=[row_spec, full((8, D))],
        out_shape=[
            jax.ShapeDtypeStruct((N, D), jnp.float32),
            jax.ShapeDtypeStruct((8, D), jnp.float32),
        ],
    )(skip, agg, deg2d, W1, b1.reshape(1, H), W2, b2.reshape(1, H),
      W3, b3.reshape(1, H), Wr, br.reshape(1, D))


def _norm_body(h_ref, sums_ref, gnw_ref, gnb_ref, gnms_ref, out_ref):
    s1 = sums_ref[0:1, :]
    s2 = sums_ref[1:2, :]
    inv_n = 1.0 / N
    mean = s1 * inv_n
    c = gnms_ref[...] * mean
    var = s2 * inv_n - 2.0 * c * mean + c * c
    scale = gnw_ref[...] * jax.lax.rsqrt(var + 1e-5)
    out_ref[...] = (h_ref[...] - c) * scale + gnb_ref[...]


def _norm(h, sums, gn_w, gn_b, gn_ms):
    grid = (N // ROW_BLK,)
    row_spec = pl.BlockSpec((ROW_BLK, D), lambda i: (i, 0))
    full = lambda shape: pl.BlockSpec(shape, lambda i: (0, 0))
    return pl.pallas_call(
        _norm_body,
        grid=grid,
        in_specs=[row_spec, full((8, D)), full((1, D)), full((1, D)),
                  full((1, D))],
        out_specs=row_spec,
        out_shape=jax.ShapeDtypeStruct((N, D), jnp.float32),
    )(h, sums, gn_w.reshape(1, D), gn_b.reshape(1, D), gn_ms.reshape(1, D))


def kernel(x, edge_index, W_skip, W_fs, W_lin, W_f, b_f, W1, b1, W2, b2,
           W3, b3, Wr, br, gn_w, gn_b, gn_ms):
    skip, xl, f2 = _pre(x, W_skip, W_fs, W_lin, W_f, b_f)
    agg_pad, deg_pad = _edge(xl, f2, edge_index[0], edge_index[1])
    deg2d = deg_pad.reshape(NPAD, 16)
    h, sums = _mlp(skip, agg_pad[:N], deg2d[:N], W1, b1, W2, b2, W3, b3,
                   Wr, br)
    return _norm(h, sums, gn_w, gn_b, gn_ms)


# speedup vs baseline: 4.5738x; 1.7803x over previous
import functools

import jax
import jax.numpy as jnp
from jax import lax
from jax.experimental import pallas as pl
from jax.experimental.pallas import tpu as pltpu
from jax.experimental.pallas import tpu_sc as plsc

N = 10000
E = 320000
D = 128
H = 256

ROW_BLK = 1000

NC = 2
NS = 16
NW = NC * NS
RNG = 320
NPAD = NW * RNG
TRASH = RNG
ACC_ROWS = RNG + 1
LIST_CAP = 16384
K = 64
LIST_ALLOC = LIST_CAP + 2 * K + 16
SCAN = 4000
PACK_SHIFT = 9
LDMASK = (1 << PACK_SHIFT) - 1


def _pre_body(x_ref, wskip_ref, wfs_ref, wlin_ref, wf_ref, bf_ref,
              skip_ref, xl_ref, f2_ref):
    xb = x_ref[...]
    fs = jnp.dot(xb, wfs_ref[...], preferred_element_type=jnp.float32)
    beta_s = fs[:, :D]
    gamma_s = fs[:, D:]
    sk = jnp.dot(xb, wskip_ref[...], preferred_element_type=jnp.float32)
    skip_ref[...] = jnp.maximum(gamma_s * sk + beta_s, 0.0)
    xl_ref[...] = jnp.dot(xb, wlin_ref[...], preferred_element_type=jnp.float32)
    f2_ref[...] = jnp.dot(xb, wf_ref[...], preferred_element_type=jnp.float32) + bf_ref[...]


def _pre(x, W_skip, W_fs, W_lin, W_f, b_f):
    grid = (N // ROW_BLK,)
    row_spec = pl.BlockSpec((ROW_BLK, D), lambda i: (i, 0))
    full = lambda shape: pl.BlockSpec(shape, lambda i: (0, 0))
    return pl.pallas_call(
        _pre_body,
        grid=grid,
        in_specs=[
            row_spec,
            full((D, D)), full((D, 2 * D)), full((D, D)), full((D, 2 * D)),
            full((1, 2 * D)),
        ],
        out_specs=[
            row_spec,
            row_spec,
            pl.BlockSpec((ROW_BLK, 2 * D), lambda i: (i, 0)),
        ],
        out_shape=[
            jax.ShapeDtypeStruct((N, D), jnp.float32),
            jax.ShapeDtypeStruct((N, D), jnp.float32),
            jax.ShapeDtypeStruct((N, 2 * D), jnp.float32),
        ],
    )(x, W_skip, W_fs, W_lin, W_f, b_f.reshape(1, 2 * D))


def _edge_body(xl_hbm, f2_hbm, esrc_hbm, edst_hbm, agg_out, deg_out,
               srcbuf, dstbuf, list_ref,
               src_idx, dst_idx, xl_buf, f2_buf,
               src_idx_b, dst_idx_b, xl_buf_b, f2_buf_b,
               acc, deg_flat, sem1, sem2, sem3, sem4):
    wid = lax.axis_index("s") * NC + lax.axis_index("c")
    lo = wid * RNG
    hi = jnp.minimum(lo + RNG, N)

    zeros16 = jnp.zeros((16,), jnp.float32)
    def _zacc(i, _):
        acc[pl.ds(i * 16, 16)] = zeros16
        return 0
    lax.fori_loop(0, (ACC_ROWS * D) // 16, _zacc, 0)

    def _zdeg(i, _):
        deg_flat[pl.ds(i * 16, 16)] = zeros16
        return 0
    lax.fori_loop(0, ACC_ROWS, _zdeg, 0)

    trash_fill = jnp.full((16,), TRASH, jnp.int32)
    def _fill(i, _):
        list_ref[pl.ds(i * 16, 16)] = trash_fill
        return 0
    lax.fori_loop(0, LIST_ALLOC // 16, _fill, 0)

    lanes = lax.iota(jnp.int32, 16)
    ones16 = jnp.full((16,), 1.0, jnp.float32)

    def _scan_chunk(c, cnt):
        off = c * SCAN
        cp1 = pltpu.async_copy(esrc_hbm.at[pl.ds(off, SCAN)], srcbuf, sem1)
        cp2 = pltpu.async_copy(edst_hbm.at[pl.ds(off, SCAN)], dstbuf, sem2)
        cp1.wait()
        cp2.wait()

        def _scan_vreg(v, cnt):
            d = dstbuf[pl.ds(v * 16, 16)]
            s = srcbuf[pl.ds(v * 16, 16)]
            m = (d >= lo) & (d < hi)
            ld = jnp.where(m, d - lo, TRASH)
            plsc.addupdate_scatter(deg_flat, [ld * 16 + lanes], ones16, mask=m)
            rank = plsc.cumsum(m.astype(jnp.int32))
            idx = cnt + rank - 1
            packed = (s << PACK_SHIFT) | ld
            plsc.store_scatter(list_ref, [idx], packed, mask=m)
            return cnt + plsc.all_reduce_population_count(m)

        return lax.fori_loop(0, SCAN // 16, _scan_vreg, cnt)

    cnt = lax.fori_loop(0, E // SCAN, _scan_chunk,
                        jnp.zeros((16,), jnp.int32))
    n_edges = cnt[0]
    n_chunks = (n_edges + (K - 1)) // K
    n_pairs = (n_chunks + 1) // 2

    jc = [lax.iota(jnp.int32, 16) + j * 16 for j in range(D // 16)]

    def _unpack_start(c, sidx, didx, xlb, f2b, semx, semf):
        base = c * K
        for v in range(K // 16):
            p = list_ref[pl.ds(base + v * 16, 16)]
            ld = p & LDMASK
            sidx[pl.ds(v * 16, 16)] = p >> PACK_SHIFT
            didx[pl.ds(v * 16, 16)] = jnp.minimum(ld + lo, N - 1)
        pltpu.async_copy(xl_hbm.at[sidx], xlb, semx)
        pltpu.async_copy(f2_hbm.at[didx], f2b, semf)

    def _wait(sidx, didx, xlb, f2b, semx, semf):
        pltpu.make_async_copy(xl_hbm.at[sidx], xlb, semx).wait()
        pltpu.make_async_copy(f2_hbm.at[didx], f2b, semf).wait()

    def _process(c, xlb, f2b):
        base = c * K

        @plsc.parallel_loop(0, K, unroll=2)
        def _edge(e):
            eb = e & ~15
            lane = e & 15
            p = list_ref[pl.ds(base + eb, 16)]
            addr = (p & LDMASK) * D
            splat = addr.at[jnp.full((16,), lane, jnp.int32)].get(
                mode="promise_in_bounds")
            for j in range(D // 16):
                xv = xlb[e, pl.ds(j * 16, 16)]
                bv = f2b[e, pl.ds(j * 16, 16)]
                gv = f2b[e, pl.ds(D + j * 16, 16)]
                plsc.addupdate_scatter(
                    acc, [splat + jc[j]],
                    jnp.maximum(gv * xv + bv, 0.0))

    _unpack_start(0, src_idx, dst_idx, xl_buf, f2_buf, sem1, sem2)

    def _pair(i, _):
        c0 = 2 * i
        _unpack_start(c0 + 1, src_idx_b, dst_idx_b, xl_buf_b, f2_buf_b,
                      sem3, sem4)
        _wait(src_idx, dst_idx, xl_buf, f2_buf, sem1, sem2)
        _process(c0, xl_buf, f2_buf)
        _unpack_start(c0 + 2, src_idx, dst_idx, xl_buf, f2_buf, sem1, sem2)
        _wait(src_idx_b, dst_idx_b, xl_buf_b, f2_buf_b, sem3, sem4)
        _process(c0 + 1, xl_buf_b, f2_buf_b)
        return 0

    lax.fori_loop(0, n_pairs, _pair, 0)
    _wait(src_idx, dst_idx, xl_buf, f2_buf, sem1, sem2)

    pltpu.sync_copy(acc.at[pl.ds(0, RNG * D)],
                    agg_out.at[pl.ds(lo * D, RNG * D)])
    pltpu.sync_copy(deg_flat.at[pl.ds(0, RNG * 16)],
                    deg_out.at[pl.ds(lo * 16, RNG * 16)])


def _edge(xl, f2, esrc, edst):
    mesh = plsc.VectorSubcoreMesh(core_axis_name="c", subcore_axis_name="s",
                                  num_cores=NC, num_subcores=NS)
    return pl.kernel(
        _edge_body,
        out_type=[
            jax.ShapeDtypeStruct((NPAD * D,), jnp.float32),
            jax.ShapeDtypeStruct((NPAD * 16,), jnp.float32),
        ],
        mesh=mesh,
        compiler_params=pltpu.CompilerParams(needs_layout_passes=False),
        scratch_types=[
            pltpu.VMEM((SCAN,), jnp.int32),
            pltpu.VMEM((SCAN,), jnp.int32),
            pltpu.VMEM((LIST_ALLOC,), jnp.int32),
            pltpu.VMEM((K,), jnp.int32),
            pltpu.VMEM((K,), jnp.int32),
            pltpu.VMEM((K, D), jnp.float32),
            pltpu.VMEM((K, 2 * D), jnp.float32),
            pltpu.VMEM((K,), jnp.int32),
            pltpu.VMEM((K,), jnp.int32),
            pltpu.VMEM((K, D), jnp.float32),
            pltpu.VMEM((K, 2 * D), jnp.float32),
            pltpu.VMEM((ACC_ROWS * D,), jnp.float32),
            pltpu.VMEM((ACC_ROWS * 16,), jnp.float32),
            pltpu.SemaphoreType.DMA,
            pltpu.SemaphoreType.DMA,
            pltpu.SemaphoreType.DMA,
            pltpu.SemaphoreType.DMA,
        ],
    )(xl, f2, esrc, edst)


def _mlp_body(skip_ref, agg_ref, deg_ref, w1_ref, b1_ref, w2_ref, b2_ref,
              w3_ref, b3_ref, wr_ref, br_ref, h_ref, sums_ref):
    deg = jnp.sum(deg_ref[...], axis=1, keepdims=True)
    deg = jnp.maximum(deg, 1.0)
    h = skip_ref[...] + agg_ref[...] / deg
    h = jnp.dot(h, w1_ref[...], preferred_element_type=jnp.float32) + b1_ref[...]
    h = jnp.where(h > 0, h, 0.01 * h)
    h = jnp.dot(h, w2_ref[...], preferred_element_type=jnp.float32) + b2_ref[...]
    h = jnp.where(h > 0, h, 0.01 * h)
    h = jnp.dot(h, w3_ref[...], preferred_element_type=jnp.float32) + b3_ref[...]
    h = jnp.where(h > 0, h, 0.01 * h)
    h = jnp.dot(h, wr_ref[...], preferred_element_type=jnp.float32) + br_ref[...]
    h_ref[...] = h

    @pl.when(pl.program_id(0) == 0)
    def _():
        sums_ref[...] = jnp.zeros_like(sums_ref)

    s1 = jnp.sum(h, axis=0, keepdims=True)
    s2 = jnp.sum(h * h, axis=0, keepdims=True)
    sums_ref[...] += jnp.concatenate(
        [s1, s2, jnp.zeros((6, D), jnp.float32)], axis=0)


def _mlp(skip, agg, deg2d, W1, b1, W2, b2, W3, b3, Wr, br):
    grid = (N // ROW_BLK,)
    row_spec = pl.BlockSpec((ROW_BLK, D), lambda i: (i, 0))
    full = lambda shape: pl.BlockSpec(shape, lambda i: (0, 0))
    return pl.pallas_call(
        _mlp_body,
        grid=grid,
        in_specs=[
            row_spec, row_spec,
            pl.BlockSpec((ROW_BLK, 16), lambda i: (i, 0)),
            full((D, H)), full((1, H)), full((H, H)), full((1, H)),
            full((H, H)), full((1, H)), full((H, D)), full((1, D)),
        ],
        out_specs=[row_spec, full((8, D))],
        out_shape=[
            jax.ShapeDtypeStruct((N, D), jnp.float32),
            jax.ShapeDtypeStruct((8, D), jnp.float32),
        ],
    )(skip, agg, deg2d, W1, b1.reshape(1, H), W2, b2.reshape(1, H),
      W3, b3.reshape(1, H), Wr, br.reshape(1, D))


def _norm_body(h_ref, sums_ref, gnw_ref, gnb_ref, gnms_ref, out_ref):
    s1 = sums_ref[0:1, :]
    s2 = sums_ref[1:2, :]
    inv_n = 1.0 / N
    mean = s1 * inv_n
    c = gnms_ref[...] * mean
    var = s2 * inv_n - 2.0 * c * mean + c * c
    scale = gnw_ref[...] * jax.lax.rsqrt(var + 1e-5)
    out_ref[...] = (h_ref[...] - c) * scale + gnb_ref[...]


def _norm(h, sums, gn_w, gn_b, gn_ms):
    grid = (N // ROW_BLK,)
    row_spec = pl.BlockSpec((ROW_BLK, D), lambda i: (i, 0))
    full = lambda shape: pl.BlockSpec(shape, lambda i: (0, 0))
    return pl.pallas_call(
        _norm_body,
        grid=grid,
        in_specs=[row_spec, full((8, D)), full((1, D)), full((1, D)),
                  full((1, D))],
        out_specs=row_spec,
        out_shape=jax.ShapeDtypeStruct((N, D), jnp.float32),
    )(h, sums, gn_w.reshape(1, D), gn_b.reshape(1, D), gn_ms.reshape(1, D))


def kernel(x, edge_index, W_skip, W_fs, W_lin, W_f, b_f, W1, b1, W2, b2,
           W3, b3, Wr, br, gn_w, gn_b, gn_ms):
    skip, xl, f2 = _pre(x, W_skip, W_fs, W_lin, W_f, b_f)
    agg_pad, deg_pad = _edge(xl, f2, edge_index[0], edge_index[1])
    agg2d = agg_pad.reshape(NPAD, D)
    deg2d = deg_pad.reshape(NPAD, 16)
    h, sums = _mlp(skip, agg2d[:N], deg2d[:N], W1, b1, W2, b2, W3, b3,
                   Wr, br)
    return _norm(h, sums, gn_w, gn_b, gn_ms)


# speedup vs baseline: 6.4134x; 1.4022x over previous
import functools

import jax
import jax.numpy as jnp
from jax import lax
from jax.experimental import pallas as pl
from jax.experimental.pallas import tpu as pltpu
from jax.experimental.pallas import tpu_sc as plsc

N = 10000
E = 320000
D = 128
H = 256

ROW_BLK = 1000

NC = 2
NS = 16
NW = NC * NS
RNG = 320
NPAD = NW * RNG
TRASH = RNG
ACC_ROWS = RNG + 1
LIST_CAP = 16384
K = 64
LIST_ALLOC = LIST_CAP + 2 * K + 16
SCAN = 3200
PACK_SHIFT = 9
LDMASK = (1 << PACK_SHIFT) - 1


def _pre_body(x_ref, wskip_ref, wfs_ref, wlin_ref, wf_ref, bf_ref,
              skip_ref, xl_ref, f2_ref):
    xb = x_ref[...]
    fs = jnp.dot(xb, wfs_ref[...], preferred_element_type=jnp.float32)
    beta_s = fs[:, :D]
    gamma_s = fs[:, D:]
    sk = jnp.dot(xb, wskip_ref[...], preferred_element_type=jnp.float32)
    skip_ref[...] = jnp.maximum(gamma_s * sk + beta_s, 0.0)
    xl_ref[...] = jnp.dot(xb, wlin_ref[...], preferred_element_type=jnp.float32)
    f2_ref[...] = jnp.dot(xb, wf_ref[...], preferred_element_type=jnp.float32) + bf_ref[...]


def _pre(x, W_skip, W_fs, W_lin, W_f, b_f):
    grid = (N // ROW_BLK,)
    row_spec = pl.BlockSpec((ROW_BLK, D), lambda i: (i, 0))
    full = lambda shape: pl.BlockSpec(shape, lambda i: (0, 0))
    return pl.pallas_call(
        _pre_body,
        grid=grid,
        in_specs=[
            row_spec,
            full((D, D)), full((D, 2 * D)), full((D, D)), full((D, 2 * D)),
            full((1, 2 * D)),
        ],
        out_specs=[
            row_spec,
            row_spec,
            pl.BlockSpec((ROW_BLK, 2 * D), lambda i: (i, 0)),
        ],
        out_shape=[
            jax.ShapeDtypeStruct((N, D), jnp.float32),
            jax.ShapeDtypeStruct((N, D), jnp.float32),
            jax.ShapeDtypeStruct((N, 2 * D), jnp.float32),
        ],
    )(x, W_skip, W_fs, W_lin, W_f, b_f.reshape(1, 2 * D))


def _edge_body(xl_hbm, f2_hbm, esrc_hbm, edst_hbm, agg_out, deg_out,
               srcbuf, dstbuf, srcbuf_b, dstbuf_b, list_ref,
               src_idx, dst_idx, xl_buf, f2_buf,
               src_idx_b, dst_idx_b, xl_buf_b, f2_buf_b,
               acc, deg_flat, sem1, sem2, sem3, sem4):
    wid = lax.axis_index("s") * NC + lax.axis_index("c")
    lo = wid * RNG
    hi = jnp.minimum(lo + RNG, N)

    zeros16 = jnp.zeros((16,), jnp.float32)
    def _zacc(i, _):
        acc[pl.ds(i * 16, 16)] = zeros16
        return 0
    lax.fori_loop(0, (ACC_ROWS * D) // 16, _zacc, 0)

    def _zdeg(i, _):
        deg_flat[pl.ds(i * 16, 16)] = zeros16
        return 0
    lax.fori_loop(0, ACC_ROWS, _zdeg, 0)

    trash_fill = jnp.full((16,), TRASH, jnp.int32)
    def _fill(i, _):
        list_ref[pl.ds(i * 16, 16)] = trash_fill
        return 0
    lax.fori_loop(0, LIST_ALLOC // 16, _fill, 0)

    lanes = lax.iota(jnp.int32, 16)
    ones16 = jnp.full((16,), 1.0, jnp.float32)

    def _scan_start(c, sb, db, semA, semB):
        off = jnp.minimum(c * SCAN, E - SCAN)
        pltpu.async_copy(esrc_hbm.at[pl.ds(off, SCAN)], sb, semA)
        pltpu.async_copy(edst_hbm.at[pl.ds(off, SCAN)], db, semB)

    def _scan_wait(c, sb, db, semA, semB):
        off = jnp.minimum(c * SCAN, E - SCAN)
        pltpu.make_async_copy(esrc_hbm.at[pl.ds(off, SCAN)], sb, semA).wait()
        pltpu.make_async_copy(edst_hbm.at[pl.ds(off, SCAN)], db, semB).wait()

    def _scan_body(sb, db, cnt0):
        @plsc.parallel_loop(0, SCAN // 16, unroll=2, carry=cnt0)
        def _scan_vreg(v, cnt):
            d = db[pl.ds(v * 16, 16)]
            s = sb[pl.ds(v * 16, 16)]
            m = (d >= lo) & (d < hi)
            ld = jnp.where(m, d - lo, TRASH)
            plsc.addupdate_scatter(deg_flat, [ld * 16 + lanes], ones16, mask=m)
            rank = plsc.cumsum(m.astype(jnp.int32))
            idx = cnt + rank - 1
            packed = (s << PACK_SHIFT) | ld
            plsc.store_scatter(list_ref, [idx], packed, mask=m)
            return cnt + plsc.all_reduce_population_count(m)
        return _scan_vreg

    _scan_start(0, srcbuf, dstbuf, sem1, sem2)

    def _scan_pair(i, cnt):
        c0 = 2 * i
        _scan_start(c0 + 1, srcbuf_b, dstbuf_b, sem3, sem4)
        _scan_wait(c0, srcbuf, dstbuf, sem1, sem2)
        cnt = _scan_body(srcbuf, dstbuf, cnt)
        _scan_start(c0 + 2, srcbuf, dstbuf, sem1, sem2)
        _scan_wait(c0 + 1, srcbuf_b, dstbuf_b, sem3, sem4)
        cnt = _scan_body(srcbuf_b, dstbuf_b, cnt)
        return cnt

    cnt = lax.fori_loop(0, E // SCAN // 2, _scan_pair,
                        jnp.zeros((16,), jnp.int32))
    _scan_wait(E // SCAN, srcbuf, dstbuf, sem1, sem2)
    n_edges = cnt[0]
    n_chunks = (n_edges + (K - 1)) // K
    n_pairs = (n_chunks + 1) // 2

    jc = [lax.iota(jnp.int32, 16) + j * 16 for j in range(D // 16)]

    def _unpack_start(c, sidx, didx, xlb, f2b, semx, semf):
        base = c * K
        for v in range(K // 16):
            p = list_ref[pl.ds(base + v * 16, 16)]
            ld = p & LDMASK
            sidx[pl.ds(v * 16, 16)] = p >> PACK_SHIFT
            didx[pl.ds(v * 16, 16)] = jnp.minimum(ld + lo, N - 1)
        pltpu.async_copy(xl_hbm.at[sidx], xlb, semx)
        pltpu.async_copy(f2_hbm.at[didx], f2b, semf)

    def _wait(sidx, didx, xlb, f2b, semx, semf):
        pltpu.make_async_copy(xl_hbm.at[sidx], xlb, semx).wait()
        pltpu.make_async_copy(f2_hbm.at[didx], f2b, semf).wait()

    def _process(c, xlb, f2b):
        base = c * K

        @plsc.parallel_loop(0, K, unroll=2)
        def _edge(e):
            eb = e & ~15
            lane = e & 15
            p = list_ref[pl.ds(base + eb, 16)]
            addr = (p & LDMASK) * D
            splat = addr.at[jnp.full((16,), lane, jnp.int32)].get(
                mode="promise_in_bounds")
            for j in range(D // 16):
                xv = xlb[e, pl.ds(j * 16, 16)]
                bv = f2b[e, pl.ds(j * 16, 16)]
                gv = f2b[e, pl.ds(D + j * 16, 16)]
                plsc.addupdate_scatter(
                    acc, [splat + jc[j]],
                    jnp.maximum(gv * xv + bv, 0.0))

    _unpack_start(0, src_idx, dst_idx, xl_buf, f2_buf, sem1, sem2)

    def _pair(i, _):
        c0 = 2 * i
        _unpack_start(c0 + 1, src_idx_b, dst_idx_b, xl_buf_b, f2_buf_b,
                      sem3, sem4)
        _wait(src_idx, dst_idx, xl_buf, f2_buf, sem1, sem2)
        _process(c0, xl_buf, f2_buf)
        _unpack_start(c0 + 2, src_idx, dst_idx, xl_buf, f2_buf, sem1, sem2)
        _wait(src_idx_b, dst_idx_b, xl_buf_b, f2_buf_b, sem3, sem4)
        _process(c0 + 1, xl_buf_b, f2_buf_b)
        return 0

    lax.fori_loop(0, n_pairs, _pair, 0)
    _wait(src_idx, dst_idx, xl_buf, f2_buf, sem1, sem2)

    pltpu.sync_copy(acc.at[pl.ds(0, RNG * D)],
                    agg_out.at[pl.ds(lo * D, RNG * D)])
    pltpu.sync_copy(deg_flat.at[pl.ds(0, RNG * 16)],
                    deg_out.at[pl.ds(lo * 16, RNG * 16)])


def _edge(xl, f2, esrc, edst):
    mesh = plsc.VectorSubcoreMesh(core_axis_name="c", subcore_axis_name="s",
                                  num_cores=NC, num_subcores=NS)
    return pl.kernel(
        _edge_body,
        out_type=[
            jax.ShapeDtypeStruct((NPAD * D,), jnp.float32),
            jax.ShapeDtypeStruct((NPAD * 16,), jnp.float32),
        ],
        mesh=mesh,
        compiler_params=pltpu.CompilerParams(needs_layout_passes=False),
        scratch_types=[
            pltpu.VMEM((SCAN,), jnp.int32),
            pltpu.VMEM((SCAN,), jnp.int32),
            pltpu.VMEM((SCAN,), jnp.int32),
            pltpu.VMEM((SCAN,), jnp.int32),
            pltpu.VMEM((LIST_ALLOC,), jnp.int32),
            pltpu.VMEM((K,), jnp.int32),
            pltpu.VMEM((K,), jnp.int32),
            pltpu.VMEM((K, D), jnp.float32),
            pltpu.VMEM((K, 2 * D), jnp.float32),
            pltpu.VMEM((K,), jnp.int32),
            pltpu.VMEM((K,), jnp.int32),
            pltpu.VMEM((K, D), jnp.float32),
            pltpu.VMEM((K, 2 * D), jnp.float32),
            pltpu.VMEM((ACC_ROWS * D,), jnp.float32),
            pltpu.VMEM((ACC_ROWS * 16,), jnp.float32),
            pltpu.SemaphoreType.DMA,
            pltpu.SemaphoreType.DMA,
            pltpu.SemaphoreType.DMA,
            pltpu.SemaphoreType.DMA,
        ],
    )(xl, f2, esrc, edst)


def _mlp_body(skip_ref, agg_ref, deg_ref, w1_ref, b1_ref, w2_ref, b2_ref,
              w3_ref, b3_ref, wr_ref, br_ref, h_ref, sums_ref):
    deg = jnp.sum(deg_ref[...], axis=1, keepdims=True)
    deg = jnp.maximum(deg, 1.0)
    h = skip_ref[...] + agg_ref[...] / deg
    h = jnp.dot(h, w1_ref[...], preferred_element_type=jnp.float32) + b1_ref[...]
    h = jnp.where(h > 0, h, 0.01 * h)
    h = jnp.dot(h, w2_ref[...], preferred_element_type=jnp.float32) + b2_ref[...]
    h = jnp.where(h > 0, h, 0.01 * h)
    h = jnp.dot(h, w3_ref[...], preferred_element_type=jnp.float32) + b3_ref[...]
    h = jnp.where(h > 0, h, 0.01 * h)
    h = jnp.dot(h, wr_ref[...], preferred_element_type=jnp.float32) + br_ref[...]
    h_ref[...] = h

    @pl.when(pl.program_id(0) == 0)
    def _():
        sums_ref[...] = jnp.zeros_like(sums_ref)

    s1 = jnp.sum(h, axis=0, keepdims=True)
    s2 = jnp.sum(h * h, axis=0, keepdims=True)
    sums_ref[...] += jnp.concatenate(
        [s1, s2, jnp.zeros((6, D), jnp.float32)], axis=0)


def _mlp(skip, agg, deg2d, W1, b1, W2, b2, W3, b3, Wr, br):
    grid = (N // ROW_BLK,)
    row_spec = pl.BlockSpec((ROW_BLK, D), lambda i: (i, 0))
    full = lambda shape: pl.BlockSpec(shape, lambda i: (0, 0))
    return pl.pallas_call(
        _mlp_body,
        grid=grid,
        in_specs=[
            row_spec, row_spec,
            pl.BlockSpec((ROW_BLK, 16), lambda i: (i, 0)),
            full((D, H)), full((1, H)), full((H, H)), full((1, H)),
            full((H, H)), full((1, H)), full((H, D)), full((1, D)),
        ],
        out_specs=[row_spec, full((8, D))],
        out_shape=[
            jax.ShapeDtypeStruct((N, D), jnp.float32),
            jax.ShapeDtypeStruct((8, D), jnp.float32),
        ],
    )(skip, agg, deg2d, W1, b1.reshape(1, H), W2, b2.reshape(1, H),
      W3, b3.reshape(1, H), Wr, br.reshape(1, D))


def _norm_body(h_ref, sums_ref, gnw_ref, gnb_ref, gnms_ref, out_ref):
    s1 = sums_ref[0:1, :]
    s2 = sums_ref[1:2, :]
    inv_n = 1.0 / N
    mean = s1 * inv_n
    c = gnms_ref[...] * mean
    var = s2 * inv_n - 2.0 * c * mean + c * c
    scale = gnw_ref[...] * jax.lax.rsqrt(var + 1e-5)
    out_ref[...] = (h_ref[...] - c) * scale + gnb_ref[...]


def _norm(h, sums, gn_w, gn_b, gn_ms):
    grid = (N // ROW_BLK,)
    row_spec = pl.BlockSpec((ROW_BLK, D), lambda i: (i, 0))
    full = lambda shape: pl.BlockSpec(shape, lambda i: (0, 0))
    return pl.pallas_call(
        _norm_body,
        grid=grid,
        in_specs=[row_spec, full((8, D)), full((1, D)), full((1, D)),
                  full((1, D))],
        out_specs=row_spec,
        out_shape=jax.ShapeDtypeStruct((N, D), jnp.float32),
    )(h, sums, gn_w.reshape(1, D), gn_b.reshape(1, D), gn_ms.reshape(1, D))


def kernel(x, edge_index, W_skip, W_fs, W_lin, W_f, b_f, W1, b1, W2, b2,
           W3, b3, Wr, br, gn_w, gn_b, gn_ms):
    skip, xl, f2 = _pre(x, W_skip, W_fs, W_lin, W_f, b_f)
    agg_pad, deg_pad = _edge(xl, f2, edge_index[0], edge_index[1])
    agg2d = agg_pad.reshape(NPAD, D)
    deg2d = deg_pad.reshape(NPAD, 16)
    h, sums = _mlp(skip, agg2d[:N], deg2d[:N], W1, b1, W2, b2, W3, b3,
                   Wr, br)
    return _norm(h, sums, gn_w, gn_b, gn_ms)


# speedup vs baseline: 6.8598x; 1.0696x over previous
import functools

import jax
import jax.numpy as jnp
from jax import lax
from jax.experimental import pallas as pl
from jax.experimental.pallas import tpu as pltpu
from jax.experimental.pallas import tpu_sc as plsc

N = 10000
E = 320000
D = 128
H = 256

ROW_BLK = 1000

NC = 2
NS = 16
NW = NC * NS
RNG = 320
NPAD = NW * RNG
TRASH = RNG
ACC_ROWS = RNG + 1
LIST_CAP = 16384
K = 64
LIST_ALLOC = LIST_CAP + 2 * K + 16
SCAN = 3200
PACK_SHIFT = 9
LDMASK = (1 << PACK_SHIFT) - 1
PKROWS = 640
PKCH = 40


def _pre_body(x_ref, wskip_ref, wfs_ref, wlin_ref, wf_ref, bf_ref,
              skip_ref, xl_ref, f2_ref):
    xb = x_ref[...]
    fs = jnp.dot(xb, wfs_ref[...], preferred_element_type=jnp.float32)
    beta_s = fs[:, :D]
    gamma_s = fs[:, D:]
    sk = jnp.dot(xb, wskip_ref[...], preferred_element_type=jnp.float32)
    skip_ref[...] = jnp.maximum(gamma_s * sk + beta_s, 0.0)
    xl_ref[...] = jnp.dot(xb, wlin_ref[...], preferred_element_type=jnp.float32)
    f2_ref[...] = jnp.dot(xb, wf_ref[...], preferred_element_type=jnp.float32) + bf_ref[...]


def _pre(x, W_skip, W_fs, W_lin, W_f, b_f):
    grid = (N // ROW_BLK,)
    row_spec = pl.BlockSpec((ROW_BLK, D), lambda i: (i, 0))
    full = lambda shape: pl.BlockSpec(shape, lambda i: (0, 0))
    return pl.pallas_call(
        _pre_body,
        grid=grid,
        in_specs=[
            row_spec,
            full((D, D)), full((D, 2 * D)), full((D, D)), full((D, 2 * D)),
            full((1, 2 * D)),
        ],
        out_specs=[
            row_spec,
            row_spec,
            pl.BlockSpec((ROW_BLK, 2 * D), lambda i: (i, 0)),
        ],
        out_shape=[
            jax.ShapeDtypeStruct((N, D), jnp.float32),
            jax.ShapeDtypeStruct((N, D), jnp.float32),
            jax.ShapeDtypeStruct((N, 2 * D), jnp.float32),
        ],
    )(x, W_skip, W_fs, W_lin, W_f, b_f.reshape(1, 2 * D))


def _edge_body(xl_hbm, f2_hbm, esrc_hbm, edst_hbm,
               agg_out, deg_out, f2p_hbm,
               srcbuf, dstbuf, srcbuf_b, dstbuf_b, list_ref,
               src_idx, dst_idx, xl_buf, f2_buf,
               src_idx_b, dst_idx_b, xl_buf_b, f2_buf_b,
               pk_f2_in, pk_f2_out,
               acc, deg_flat, sem1, sem2, sem3, sem4):
    cid = lax.axis_index("c")
    sid = lax.axis_index("s")
    wid = sid * NC + cid
    lo = wid * RNG
    hi = jnp.minimum(lo + RNG, N)
    plane = cid * N

    r0 = sid * PKROWS
    n_rows = jnp.minimum(PKROWS, N - r0)
    n_pk = (n_rows + PKCH - 1) // PKCH

    def _pack_chunk(c, _):
        rc = r0 + c * PKCH
        pltpu.sync_copy(f2_hbm.at[pl.ds(rc, PKCH)], pk_f2_in)

        @plsc.parallel_loop(0, PKCH, unroll=2)
        def _row(r):
            for j in range(D // 16):
                bta = pk_f2_in[r, pl.ds(j * 16, 16)]
                gma = pk_f2_in[r, pl.ds(D + j * 16, 16)]
                pk = plsc.pack(bta, gma, format=plsc.PackFormat.INTERLEAVED)
                pk_f2_out[r, pl.ds(j * 16, 16)] = plsc.bitcast(pk, jnp.int32)

        pltpu.sync_copy(pk_f2_out, f2p_hbm.at[pl.ds(plane + rc, PKCH)])
        return 0

    lax.fori_loop(0, n_pk, _pack_chunk, 0)
    plsc.subcore_barrier()

    zeros16 = jnp.zeros((16,), jnp.float32)
    def _zacc(i, _):
        acc[pl.ds(i * 16, 16)] = zeros16
        return 0
    lax.fori_loop(0, (ACC_ROWS * D) // 16, _zacc, 0)

    def _zdeg(i, _):
        deg_flat[pl.ds(i * 16, 16)] = zeros16
        return 0
    lax.fori_loop(0, ACC_ROWS, _zdeg, 0)

    trash_fill = jnp.full((16,), TRASH, jnp.int32)
    def _fill(i, _):
        list_ref[pl.ds(i * 16, 16)] = trash_fill
        return 0
    lax.fori_loop(0, LIST_ALLOC // 16, _fill, 0)

    lanes = lax.iota(jnp.int32, 16)
    ones16 = jnp.full((16,), 1.0, jnp.float32)

    def _scan_start(c, sb, db, semA, semB):
        off = jnp.minimum(c * SCAN, E - SCAN)
        pltpu.async_copy(esrc_hbm.at[pl.ds(off, SCAN)], sb, semA)
        pltpu.async_copy(edst_hbm.at[pl.ds(off, SCAN)], db, semB)

    def _scan_wait(c, sb, db, semA, semB):
        off = jnp.minimum(c * SCAN, E - SCAN)
        pltpu.make_async_copy(esrc_hbm.at[pl.ds(off, SCAN)], sb, semA).wait()
        pltpu.make_async_copy(edst_hbm.at[pl.ds(off, SCAN)], db, semB).wait()

    def _scan_body(sb, db, cnt0):
        @plsc.parallel_loop(0, SCAN // 16, unroll=2, carry=cnt0)
        def _scan_vreg(v, cnt):
            d = db[pl.ds(v * 16, 16)]
            s = sb[pl.ds(v * 16, 16)]
            m = (d >= lo) & (d < hi)
            ld = jnp.where(m, d - lo, TRASH)
            plsc.addupdate_scatter(deg_flat, [ld * 16 + lanes], ones16, mask=m)
            rank = plsc.cumsum(m.astype(jnp.int32))
            idx = cnt + rank - 1
            packed = (s << PACK_SHIFT) | ld
            plsc.store_scatter(list_ref, [idx], packed, mask=m)
            return cnt + plsc.all_reduce_population_count(m)
        return _scan_vreg

    _scan_start(0, srcbuf, dstbuf, sem1, sem2)

    def _scan_pair(i, cnt):
        c0 = 2 * i
        _scan_start(c0 + 1, srcbuf_b, dstbuf_b, sem3, sem4)
        _scan_wait(c0, srcbuf, dstbuf, sem1, sem2)
        cnt = _scan_body(srcbuf, dstbuf, cnt)
        _scan_start(c0 + 2, srcbuf, dstbuf, sem1, sem2)
        _scan_wait(c0 + 1, srcbuf_b, dstbuf_b, sem3, sem4)
        cnt = _scan_body(srcbuf_b, dstbuf_b, cnt)
        return cnt

    cnt = lax.fori_loop(0, E // SCAN // 2, _scan_pair,
                        jnp.zeros((16,), jnp.int32))
    _scan_wait(E // SCAN, srcbuf, dstbuf, sem1, sem2)
    n_edges = cnt[0]
    n_chunks = (n_edges + (K - 1)) // K
    n_pairs = (n_chunks + 1) // 2

    jc = [lax.iota(jnp.int32, 16) + j * 16 for j in range(D // 16)]

    def _unpack_start(c, sidx, didx, xlb, f2b, semx, semf):
        base = c * K
        for v in range(K // 16):
            p = list_ref[pl.ds(base + v * 16, 16)]
            ld = p & LDMASK
            sidx[pl.ds(v * 16, 16)] = p >> PACK_SHIFT
            didx[pl.ds(v * 16, 16)] = jnp.minimum(ld + lo, N - 1) + plane
        pltpu.async_copy(xl_hbm.at[sidx], xlb, semx)
        pltpu.async_copy(f2p_hbm.at[didx], f2b, semf)

    def _wait(sidx, didx, xlb, f2b, semx, semf):
        pltpu.make_async_copy(xl_hbm.at[sidx], xlb, semx).wait()
        pltpu.make_async_copy(f2p_hbm.at[didx], f2b, semf).wait()

    def _process(c, xlb, f2b):
        base = c * K

        @plsc.parallel_loop(0, K, unroll=2)
        def _edge(e):
            eb = e & ~15
            lane = e & 15
            p = list_ref[pl.ds(base + eb, 16)]
            addr = (p & LDMASK) * D
            splat = addr.at[jnp.full((16,), lane, jnp.int32)].get(
                mode="promise_in_bounds")
            for j in range(D // 16):
                xv = xlb[e, pl.ds(j * 16, 16)]
                w = plsc.bitcast(f2b[e, pl.ds(j * 16, 16)], jnp.bfloat16)
                bv, gv = plsc.unpack(w, format=plsc.PackFormat.INTERLEAVED)
                plsc.addupdate_scatter(
                    acc, [splat + jc[j]],
                    jnp.maximum(gv * xv + bv, 0.0))

    _unpack_start(0, src_idx, dst_idx, xl_buf, f2_buf, sem1, sem2)

    def _pair(i, _):
        c0 = 2 * i
        _unpack_start(c0 + 1, src_idx_b, dst_idx_b, xl_buf_b, f2_buf_b,
                      sem3, sem4)
        _wait(src_idx, dst_idx, xl_buf, f2_buf, sem1, sem2)
        _process(c0, xl_buf, f2_buf)
        _unpack_start(c0 + 2, src_idx, dst_idx, xl_buf, f2_buf, sem1, sem2)
        _wait(src_idx_b, dst_idx_b, xl_buf_b, f2_buf_b, sem3, sem4)
        _process(c0 + 1, xl_buf_b, f2_buf_b)
        return 0

    lax.fori_loop(0, n_pairs, _pair, 0)
    _wait(src_idx, dst_idx, xl_buf, f2_buf, sem1, sem2)

    pltpu.sync_copy(acc.at[pl.ds(0, RNG * D)],
                    agg_out.at[pl.ds(lo * D, RNG * D)])
    pltpu.sync_copy(deg_flat.at[pl.ds(0, RNG * 16)],
                    deg_out.at[pl.ds(lo * 16, RNG * 16)])


def _edge(xl, f2, esrc, edst):
    mesh = plsc.VectorSubcoreMesh(core_axis_name="c", subcore_axis_name="s",
                                  num_cores=NC, num_subcores=NS)
    return pl.kernel(
        _edge_body,
        out_type=[
            jax.ShapeDtypeStruct((NPAD * D,), jnp.float32),
            jax.ShapeDtypeStruct((NPAD * 16,), jnp.float32),
            jax.ShapeDtypeStruct((2 * N, D), jnp.int32),
        ],
        mesh=mesh,
        compiler_params=pltpu.CompilerParams(needs_layout_passes=False),
        scratch_types=[
            pltpu.VMEM((SCAN,), jnp.int32),
            pltpu.VMEM((SCAN,), jnp.int32),
            pltpu.VMEM((SCAN,), jnp.int32),
            pltpu.VMEM((SCAN,), jnp.int32),
            pltpu.VMEM((LIST_ALLOC,), jnp.int32),
            pltpu.VMEM((K,), jnp.int32),
            pltpu.VMEM((K,), jnp.int32),
            pltpu.VMEM((K, D), jnp.float32),
            pltpu.VMEM((K, D), jnp.int32),
            pltpu.VMEM((K,), jnp.int32),
            pltpu.VMEM((K,), jnp.int32),
            pltpu.VMEM((K, D), jnp.float32),
            pltpu.VMEM((K, D), jnp.int32),
            pltpu.VMEM((PKCH, 2 * D), jnp.float32),
            pltpu.VMEM((PKCH, D), jnp.int32),
            pltpu.VMEM((ACC_ROWS * D,), jnp.float32),
            pltpu.VMEM((ACC_ROWS * 16,), jnp.float32),
            pltpu.SemaphoreType.DMA,
            pltpu.SemaphoreType.DMA,
            pltpu.SemaphoreType.DMA,
            pltpu.SemaphoreType.DMA,
        ],
    )(xl, f2, esrc, edst)


def _mlp_body(skip_ref, agg_ref, deg_ref, w1_ref, b1_ref, w2_ref, b2_ref,
              w3_ref, b3_ref, wr_ref, br_ref, h_ref, sums_ref):
    deg = jnp.sum(deg_ref[...], axis=1, keepdims=True)
    deg = jnp.maximum(deg, 1.0)
    h = skip_ref[...] + agg_ref[...] / deg
    h = jnp.dot(h, w1_ref[...], preferred_element_type=jnp.float32) + b1_ref[...]
    h = jnp.where(h > 0, h, 0.01 * h)
    h = jnp.dot(h, w2_ref[...], preferred_element_type=jnp.float32) + b2_ref[...]
    h = jnp.where(h > 0, h, 0.01 * h)
    h = jnp.dot(h, w3_ref[...], preferred_element_type=jnp.float32) + b3_ref[...]
    h = jnp.where(h > 0, h, 0.01 * h)
    h = jnp.dot(h, wr_ref[...], preferred_element_type=jnp.float32) + br_ref[...]
    h_ref[...] = h

    @pl.when(pl.program_id(0) == 0)
    def _():
        sums_ref[...] = jnp.zeros_like(sums_ref)

    s1 = jnp.sum(h, axis=0, keepdims=True)
    s2 = jnp.sum(h * h, axis=0, keepdims=True)
    sums_ref[...] += jnp.concatenate(
        [s1, s2, jnp.zeros((6, D), jnp.float32)], axis=0)


def _mlp(skip, agg, deg2d, W1, b1, W2, b2, W3, b3, Wr, br):
    grid = (N // ROW_BLK,)
    row_spec = pl.BlockSpec((ROW_BLK, D), lambda i: (i, 0))
    full = lambda shape: pl.BlockSpec(shape, lambda i: (0, 0))
    return pl.pallas_call(
        _mlp_body,
        grid=grid,
        in_specs=[
            row_spec, row_spec,
            pl.BlockSpec((ROW_BLK, 16), lambda i: (i, 0)),
            full((D, H)), full((1, H)), full((H, H)), full((1, H)),
            full((H, H)), full((1, H)), full((H, D)), full((1, D)),
        ],
        out_specs=[row_spec, full((8, D))],
        out_shape=[
            jax.ShapeDtypeStruct((N, D), jnp.float32),
            jax.ShapeDtypeStruct((8, D), jnp.float32),
        ],
    )(skip, agg, deg2d, W1, b1.reshape(1, H), W2, b2.reshape(1, H),
      W3, b3.reshape(1, H), Wr, br.reshape(1, D))


def _norm_body(h_ref, sums_ref, gnw_ref, gnb_ref, gnms_ref, out_ref):
    s1 = sums_ref[0:1, :]
    s2 = sums_ref[1:2, :]
    inv_n = 1.0 / N
    mean = s1 * inv_n
    c = gnms_ref[...] * mean
    var = s2 * inv_n - 2.0 * c * mean + c * c
    scale = gnw_ref[...] * jax.lax.rsqrt(var + 1e-5)
    out_ref[...] = (h_ref[...] - c) * scale + gnb_ref[...]


def _norm(h, sums, gn_w, gn_b, gn_ms):
    grid = (N // ROW_BLK,)
    row_spec = pl.BlockSpec((ROW_BLK, D), lambda i: (i, 0))
    full = lambda shape: pl.BlockSpec(shape, lambda i: (0, 0))
    return pl.pallas_call(
        _norm_body,
        grid=grid,
        in_specs=[row_spec, full((8, D)), full((1, D)), full((1, D)),
                  full((1, D))],
        out_specs=row_spec,
        out_shape=jax.ShapeDtypeStruct((N, D), jnp.float32),
    )(h, sums, gn_w.reshape(1, D), gn_b.reshape(1, D), gn_ms.reshape(1, D))


def kernel(x, edge_index, W_skip, W_fs, W_lin, W_f, b_f, W1, b1, W2, b2,
           W3, b3, Wr, br, gn_w, gn_b, gn_ms):
    skip, xl, f2 = _pre(x, W_skip, W_fs, W_lin, W_f, b_f)
    agg_pad, deg_pad, _f2p = _edge(xl, f2, edge_index[0], edge_index[1])
    agg2d = agg_pad.reshape(NPAD, D)
    deg2d = deg_pad.reshape(NPAD, 16)
    h, sums = _mlp(skip, agg2d[:N], deg2d[:N], W1, b1, W2, b2, W3, b3,
                   Wr, br)
    return _norm(h, sums, gn_w, gn_b, gn_ms)
